# Initial kernel scaffold; baseline (speedup 1.0000x reference)
import jax
import jax.numpy as jnp
from jax.experimental import pallas as pl


def kernel(src, dst, ufeats, mfeats, Wn_u, Ws_u, b_u, Wn_m, Ws_m, b_m, decoders):
    raise NotImplementedError("write your pallas kernel here")


# speedup vs baseline: 1.3336x; 1.3336x over previous
import functools

import jax
import jax.numpy as jnp
from jax import lax
from jax.experimental import pallas as pl
from jax.experimental.pallas import tpu as pltpu
from jax.experimental.pallas import tpu_sc as plsc

N_USERS = 50000
N_MOVIES = 10000
DU = 128
DM = 64
H = 64
R = 5
E = 20000
D = R * H
E2 = 20480

INTERPRET = False


def _make_agg(width, nseg, n_passes):
    nseg_pc = nseg // 2
    if n_passes == 1:
        sizes = [nseg_pc]
    else:
        assert n_passes == 2
        s0 = (nseg_pc // 2 + 7) // 8 * 8
        sizes = [s0, nseg_pc - s0]
    trash = max(sizes)
    acc_rows = trash + 8
    mesh = plsc.VectorSubcoreMesh(core_axis_name="c", subcore_axis_name="s")

    @functools.partial(
        pl.kernel,
        out_type=[jax.ShapeDtypeStruct((R, nseg, width), jnp.float32),
                  jax.ShapeDtypeStruct((R, nseg, 16), jnp.float32)],
        mesh=mesh,
        scratch_types=[
            pltpu.VMEM_SHARED((acc_rows, width), jnp.float32),
            pltpu.VMEM_SHARED((acc_rows, 16), jnp.float32),
            pltpu.VMEM((10, 128), jnp.int32),
            pltpu.VMEM((10, 128), jnp.int32),
            pltpu.VMEM((128, width), jnp.float32),
            pltpu.VMEM((128, 16), jnp.float32),
            pltpu.SemaphoreType.DMA,
        ],
        compiler_params=pltpu.CompilerParams(use_tc_tiling_on_sc=False, needs_layout_passes=False),
    )
    def agg(table, gidx, sidx, zerosw, zeros16, out_sum, out_deg,
            acc_sh, dacc_sh, gidx_v, scidx_v, rows_v, ones_v, sem):
        ci = lax.axis_index("c")
        sid = lax.axis_index("s")
        base = ci * nseg_pc

        def initones(i, c):
            ones_v[i, :] = jnp.full((16,), 1.0, jnp.float32)
            return c
        lax.fori_loop(0, 128, initones, 0)

        for r in range(R):
            for p, size in enumerate(sizes):
                pbase = base + p * sizes[0]
                zr = (size // 8) // 8 * 8
                tail = size - 8 * zr

                @pl.when(sid < 8)
                def _():
                    pltpu.sync_copy(zerosw.at[pl.ds(0, zr)],
                                    acc_sh.at[pl.ds(sid * zr, zr)])
                    pltpu.sync_copy(zeros16.at[pl.ds(0, zr)],
                                    dacc_sh.at[pl.ds(sid * zr, zr)])
                if tail:
                    @pl.when(sid == 8)
                    def _():
                        pltpu.sync_copy(zerosw.at[pl.ds(0, tail)],
                                        acc_sh.at[pl.ds(8 * zr, tail)])
                        pltpu.sync_copy(zeros16.at[pl.ds(0, tail)],
                                        dacc_sh.at[pl.ds(8 * zr, tail)])
                plsc.subcore_barrier()

                pltpu.sync_copy(gidx.at[r, sid], gidx_v)
                pltpu.sync_copy(sidx.at[r, sid], scidx_v)

                def fix(j, c):
                    for k in range(8):
                        v = scidx_v[j, pl.ds(k * 16, 16)]
                        li = v - pbase
                        oob = (li < 0) | (li >= size)
                        scidx_v[j, pl.ds(k * 16, 16)] = jnp.where(oob, trash, li)
                    return c
                lax.fori_loop(0, 10, fix, 0)

                def chunk(j, c):
                    pltpu.async_copy(table.at[gidx_v.at[j]], rows_v, sem).wait()
                    pltpu.sync_copy(rows_v, acc_sh.at[scidx_v.at[j]], add=True)
                    pltpu.sync_copy(ones_v, dacc_sh.at[scidx_v.at[j]], add=True)
                    return c
                lax.fori_loop(0, 10, chunk, 0)
                plsc.subcore_barrier()

                @pl.when(sid < 8)
                def _():
                    off = pbase + sid * zr
                    pltpu.sync_copy(acc_sh.at[pl.ds(sid * zr, zr)],
                                    out_sum.at[r, pl.ds(off, zr)])
                    pltpu.sync_copy(dacc_sh.at[pl.ds(sid * zr, zr)],
                                    out_deg.at[r, pl.ds(off, zr)])
                if tail:
                    @pl.when(sid == 8)
                    def _():
                        off = pbase + 8 * zr
                        pltpu.sync_copy(acc_sh.at[pl.ds(8 * zr, tail)],
                                        out_sum.at[r, pl.ds(off, tail)])
                        pltpu.sync_copy(dacc_sh.at[pl.ds(8 * zr, tail)],
                                        out_deg.at[r, pl.ds(off, tail)])
                plsc.subcore_barrier()

    return agg


_CH = 16
_EPW = E2 // 32
_NCH = _EPW // _CH


def _make_decode():
    mesh = plsc.VectorSubcoreMesh(core_axis_name="c", subcore_axis_name="s")

    @functools.partial(
        pl.kernel,
        out_type=jax.ShapeDtypeStruct((R, E2, 16), jnp.float32),
        mesh=mesh,
        scratch_types=[
            pltpu.VMEM((_EPW,), jnp.int32),
            pltpu.VMEM((_EPW,), jnp.int32),
            pltpu.VMEM((_CH, D), jnp.float32),
            pltpu.VMEM((_CH, R * D), jnp.float32),
            pltpu.VMEM((_EPW, 16), jnp.float32),
            pltpu.SemaphoreType.DMA,
            pltpu.SemaphoreType.DMA,
        ],
        compiler_params=pltpu.CompilerParams(use_tc_tiling_on_sc=False, needs_layout_passes=False),
    )
    def decode(res_user, qall, uidx, qidx, out,
               uidx_v, qidx_v, ubuf, qbuf, dots_v, usem, qsem):
        ci = lax.axis_index("c")
        sid = lax.axis_index("s")
        w = sid * 2 + ci
        for r in range(R):
            pltpu.sync_copy(uidx.at[r, w], uidx_v)
            pltpu.sync_copy(qidx.at[r, w], qidx_v)

            def chunk(j, c):
                pltpu.async_copy(
                    res_user.at[uidx_v.at[pl.ds(j * _CH, _CH)]], ubuf, usem).wait()
                pltpu.async_copy(
                    qall.at[qidx_v.at[pl.ds(j * _CH, _CH)]], qbuf, qsem).wait()

                def edge(e, c2):
                    uk = [ubuf[e, pl.ds(k * 16, 16)] for k in range(D // 16)]
                    lanes = lax.iota(jnp.int32, 16)
                    row = jnp.full((16,), 0.0, jnp.float32)
                    for s in range(R):
                        acc = uk[0] * qbuf[e, pl.ds(s * D, 16)]
                        for k in range(1, D // 16):
                            acc = acc + uk[k] * qbuf[e, pl.ds(s * D + k * 16, 16)]
                        dval = jnp.full((16,), jnp.sum(acc), jnp.float32)
                        row = jnp.where(lanes == s, dval, row)
                    dots_v[j * _CH + e, :] = row
                    return c2
                lax.fori_loop(0, _CH, edge, 0)
                return c
            lax.fori_loop(0, _NCH, chunk, 0)
            pltpu.sync_copy(dots_v, out.at[r, pl.ds(w * _EPW, _EPW)])

    return decode


def _sage_body(nsum_ref, deg_ref, feat_ref, Wn_ref, Ws_ref, b_ref, out_ref):
    feat = feat_ref[...]
    for r in range(R):
        deg = jnp.maximum(deg_ref[r, :, 0:1], 1.0)
        nmean = nsum_ref[r] / deg
        acc = jnp.dot(nmean, Wn_ref[r], preferred_element_type=jnp.float32)
        acc = acc + jnp.dot(feat, Ws_ref[r], preferred_element_type=jnp.float32)
        acc = acc + b_ref[0, r * H:(r + 1) * H][None, :]
        out_ref[:, r * H:(r + 1) * H] = jnp.maximum(acc, 0.0)


def _sage_transform(nsum, deg, feat, Wn, Ws, b, n_rows, blk):
    dn = nsum.shape[-1]
    df = feat.shape[-1]
    return pl.pallas_call(
        _sage_body,
        grid=(n_rows // blk,),
        in_specs=[
            pl.BlockSpec((R, blk, dn), lambda i: (0, i, 0)),
            pl.BlockSpec((R, blk, 16), lambda i: (0, i, 0)),
            pl.BlockSpec((blk, df), lambda i: (i, 0)),
            pl.BlockSpec((R, dn, H), lambda i: (0, 0, 0)),
            pl.BlockSpec((R, df, H), lambda i: (0, 0, 0)),
            pl.BlockSpec((1, R * H), lambda i: (0, 0)),
        ],
        out_specs=pl.BlockSpec((blk, R * H), lambda i: (i, 0)),
        out_shape=jax.ShapeDtypeStruct((n_rows, R * H), jnp.float32),
        interpret=INTERPRET,
    )(nsum, deg, feat, Wn, Ws, b)


def _qall_body(rm_ref, bdec_ref, out_ref):
    out_ref[...] = jnp.dot(rm_ref[...], bdec_ref[...],
                           preferred_element_type=jnp.float32)


def _qall(res_movie, bdec):
    blk = 1000
    return pl.pallas_call(
        _qall_body,
        grid=(N_MOVIES // blk,),
        in_specs=[
            pl.BlockSpec((blk, D), lambda i: (i, 0)),
            pl.BlockSpec((D, R * D), lambda i: (0, 0)),
        ],
        out_specs=pl.BlockSpec((blk, R * D), lambda i: (i, 0)),
        out_shape=jax.ShapeDtypeStruct((N_MOVIES, R * D), jnp.float32),
        interpret=INTERPRET,
    )(res_movie, bdec)


def _lsm_body(dots_ref, out_ref):
    x = dots_ref[0]
    lane = lax.broadcasted_iota(jnp.int32, x.shape, 1)
    valid = lane < R
    xm = jnp.where(valid, x, -jnp.inf)
    m = jnp.max(xm, axis=1, keepdims=True)
    ex = jnp.where(valid, jnp.exp(x - m), 0.0)
    lse = jnp.log(jnp.sum(ex, axis=1, keepdims=True))
    out_ref[0] = jnp.where(valid, x - m - lse, 0.0)


def _log_softmax(dots):
    blk = 2048
    return pl.pallas_call(
        _lsm_body,
        grid=(R, E2 // blk),
        in_specs=[pl.BlockSpec((1, blk, 16), lambda r, i: (r, i, 0))],
        out_specs=pl.BlockSpec((1, blk, 16), lambda r, i: (r, i, 0)),
        out_shape=jax.ShapeDtypeStruct((R, E2, 16), jnp.float32),
        interpret=INTERPRET,
    )(dots)


def _pad_edges(idx, fill):
    pad = jnp.full((R, E2 - E), fill, jnp.int32)
    return jnp.concatenate([idx, pad], axis=1)


def kernel(src, dst, ufeats, mfeats, Wn_u, Ws_u, b_u, Wn_m, Ws_m, b_m, decoders):
    src0 = _pad_edges(src, 0)
    dst0 = _pad_edges(dst, 0)
    srcm1 = _pad_edges(src, -1)
    dstm1 = _pad_edges(dst, -1)

    agg_m = _make_agg(DU, N_MOVIES, 1)
    nm_sum, deg_m = agg_m(
        ufeats,
        src0.reshape(R, 16, 10, 128), dstm1.reshape(R, 16, 10, 128),
        jnp.zeros((N_MOVIES // 16, DU), jnp.float32),
        jnp.zeros((N_MOVIES // 16, 16), jnp.float32))
    agg_u = _make_agg(DM, N_USERS, 2)
    nu_sum, deg_u = agg_u(
        mfeats,
        dst0.reshape(R, 16, 10, 128), srcm1.reshape(R, 16, 10, 128),
        jnp.zeros((N_USERS // 16, DM), jnp.float32),
        jnp.zeros((N_USERS // 16, 16), jnp.float32))

    res_movie = _sage_transform(nm_sum, deg_m, mfeats, Wn_u, Ws_u,
                                b_u.reshape(1, R * H), N_MOVIES, 1000)
    res_user = _sage_transform(nu_sum, deg_u, ufeats, Wn_m, Ws_m,
                               b_m.reshape(1, R * H), N_USERS, 1000)
    bdec = decoders.transpose(2, 0, 1).reshape(D, R * D)
    qall = _qall(res_movie, bdec)

    decode = _make_decode()
    dots = decode(res_user, qall, src0.reshape(R, 32, _EPW),
                  dst0.reshape(R, 32, _EPW))

    out = _log_softmax(dots)
    return tuple(out[r, :E, :R] for r in range(R))


# speedup vs baseline: 1.7947x; 1.3458x over previous
import functools

import jax
import jax.numpy as jnp
from jax import lax
from jax.experimental import pallas as pl
from jax.experimental.pallas import tpu as pltpu
from jax.experimental.pallas import tpu_sc as plsc

N_USERS = 50000
N_MOVIES = 10000
DU = 128
DM = 64
H = 64
R = 5
E = 20000
D = R * H
E2 = 20480

INTERPRET = False


def _make_agg(width, nseg, n_passes):
    nseg_pc = nseg // 2
    if n_passes == 1:
        sizes = [nseg_pc]
    else:
        assert n_passes == 2
        s0 = (nseg_pc // 2 + 7) // 8 * 8
        sizes = [s0, nseg_pc - s0]
    trash = max(sizes)
    acc_rows = trash + 8
    mesh = plsc.VectorSubcoreMesh(core_axis_name="c", subcore_axis_name="s")

    @functools.partial(
        pl.kernel,
        out_type=[jax.ShapeDtypeStruct((R, nseg, width), jnp.float32),
                  jax.ShapeDtypeStruct((R, nseg, 16), jnp.float32)],
        mesh=mesh,
        scratch_types=[
            pltpu.VMEM_SHARED((acc_rows, width), jnp.float32),
            pltpu.VMEM_SHARED((acc_rows, 16), jnp.float32),
            pltpu.VMEM((10, 128), jnp.int32),
            pltpu.VMEM((10, 128), jnp.int32),
            pltpu.VMEM((128, width), jnp.float32),
            pltpu.VMEM((128, width), jnp.float32),
            pltpu.VMEM((128, 16), jnp.float32),
            pltpu.SemaphoreType.DMA,
            pltpu.SemaphoreType.DMA,
        ],
        compiler_params=pltpu.CompilerParams(use_tc_tiling_on_sc=False, needs_layout_passes=False),
    )
    def agg(table, gidx, sidx, zerosw, zeros16, out_sum, out_deg,
            acc_sh, dacc_sh, gidx_v, scidx_v, rows_a, rows_b, ones_v,
            gsem_a, gsem_b):
        ci = lax.axis_index("c")
        sid = lax.axis_index("s")
        base = ci * nseg_pc

        def initones(i, c):
            ones_v[i, :] = jnp.full((16,), 1.0, jnp.float32)
            return c
        lax.fori_loop(0, 128, initones, 0)

        for r in range(R):
            for p, size in enumerate(sizes):
                pbase = base + p * sizes[0]
                zr = (size // 8) // 8 * 8
                tail = size - 8 * zr

                @pl.when(sid < 8)
                def _():
                    pltpu.sync_copy(zerosw.at[pl.ds(0, zr)],
                                    acc_sh.at[pl.ds(sid * zr, zr)])
                    pltpu.sync_copy(zeros16.at[pl.ds(0, zr)],
                                    dacc_sh.at[pl.ds(sid * zr, zr)])
                if tail:
                    @pl.when(sid == 8)
                    def _():
                        pltpu.sync_copy(zerosw.at[pl.ds(0, tail)],
                                        acc_sh.at[pl.ds(8 * zr, tail)])
                        pltpu.sync_copy(zeros16.at[pl.ds(0, tail)],
                                        dacc_sh.at[pl.ds(8 * zr, tail)])
                plsc.subcore_barrier()

                pltpu.sync_copy(gidx.at[r, sid], gidx_v)
                pltpu.sync_copy(sidx.at[r, sid], scidx_v)

                def fix(j, c):
                    for k in range(8):
                        v = scidx_v[j, pl.ds(k * 16, 16)]
                        li = v - pbase
                        oob = (li < 0) | (li >= size)
                        scidx_v[j, pl.ds(k * 16, 16)] = jnp.where(oob, trash, li)
                    return c
                lax.fori_loop(0, 10, fix, 0)

                pltpu.async_copy(table.at[gidx_v.at[0]], rows_a, gsem_a)

                def pair(pp, c):
                    ja = 2 * pp
                    pltpu.async_copy(table.at[gidx_v.at[ja + 1]], rows_b, gsem_b)
                    pltpu.make_async_copy(table, rows_a, gsem_a).wait()
                    pltpu.sync_copy(rows_a, acc_sh.at[scidx_v.at[ja]], add=True)
                    pltpu.sync_copy(ones_v, dacc_sh.at[scidx_v.at[ja]], add=True)

                    @pl.when(pp < 4)
                    def _():
                        pltpu.async_copy(table.at[gidx_v.at[ja + 2]],
                                         rows_a, gsem_a)
                    pltpu.make_async_copy(table, rows_b, gsem_b).wait()
                    pltpu.sync_copy(rows_b, acc_sh.at[scidx_v.at[ja + 1]], add=True)
                    pltpu.sync_copy(ones_v, dacc_sh.at[scidx_v.at[ja + 1]], add=True)
                    return c
                lax.fori_loop(0, 5, pair, 0)
                plsc.subcore_barrier()

                @pl.when(sid < 8)
                def _():
                    off = pbase + sid * zr
                    pltpu.sync_copy(acc_sh.at[pl.ds(sid * zr, zr)],
                                    out_sum.at[r, pl.ds(off, zr)])
                    pltpu.sync_copy(dacc_sh.at[pl.ds(sid * zr, zr)],
                                    out_deg.at[r, pl.ds(off, zr)])
                if tail:
                    @pl.when(sid == 8)
                    def _():
                        off = pbase + 8 * zr
                        pltpu.sync_copy(acc_sh.at[pl.ds(8 * zr, tail)],
                                        out_sum.at[r, pl.ds(off, tail)])
                        pltpu.sync_copy(dacc_sh.at[pl.ds(8 * zr, tail)],
                                        out_deg.at[r, pl.ds(off, tail)])
                plsc.subcore_barrier()

    return agg


_CH = 16
_EPW = E2 // 32
_NCH = _EPW // _CH


def _make_decode():
    mesh = plsc.VectorSubcoreMesh(core_axis_name="c", subcore_axis_name="s")

    @functools.partial(
        pl.kernel,
        out_type=jax.ShapeDtypeStruct((R, E2, 16), jnp.float32),
        mesh=mesh,
        scratch_types=[
            pltpu.VMEM((_EPW,), jnp.int32),
            pltpu.VMEM((_EPW,), jnp.int32),
            pltpu.VMEM((_CH, D), jnp.float32),
            pltpu.VMEM((_CH, D), jnp.float32),
            pltpu.VMEM((_CH, R * D), jnp.float32),
            pltpu.VMEM((_CH, R * D), jnp.float32),
            pltpu.VMEM((_EPW, 16), jnp.float32),
            pltpu.SemaphoreType.DMA,
            pltpu.SemaphoreType.DMA,
            pltpu.SemaphoreType.DMA,
            pltpu.SemaphoreType.DMA,
        ],
        compiler_params=pltpu.CompilerParams(use_tc_tiling_on_sc=False, needs_layout_passes=False),
    )
    def decode(res_user, qall, uidx, qidx, out,
               uidx_v, qidx_v, ubuf_a, ubuf_b, qbuf_a, qbuf_b, dots_v,
               usem_a, usem_b, qsem_a, qsem_b):
        ci = lax.axis_index("c")
        sid = lax.axis_index("s")
        w = sid * 2 + ci

        def start(j, ubuf, qbuf, usem, qsem):
            pltpu.async_copy(
                res_user.at[uidx_v.at[pl.ds(j * _CH, _CH)]], ubuf, usem)
            pltpu.async_copy(
                qall.at[qidx_v.at[pl.ds(j * _CH, _CH)]], qbuf, qsem)

        def waitbufs(ubuf, qbuf, usem, qsem):
            pltpu.make_async_copy(res_user, ubuf, usem).wait()
            pltpu.make_async_copy(qall, qbuf, qsem).wait()

        def compute(j, ubuf, qbuf):
            def edge(e, c2):
                uk = [ubuf[e, pl.ds(k * 16, 16)] for k in range(D // 16)]
                lanes = lax.iota(jnp.int32, 16)
                row = jnp.full((16,), 0.0, jnp.float32)
                for s in range(R):
                    acc = uk[0] * qbuf[e, pl.ds(s * D, 16)]
                    for k in range(1, D // 16):
                        acc = acc + uk[k] * qbuf[e, pl.ds(s * D + k * 16, 16)]
                    dval = jnp.full((16,), jnp.sum(acc), jnp.float32)
                    row = jnp.where(lanes == s, dval, row)
                dots_v[j * _CH + e, :] = row
                return c2
            lax.fori_loop(0, _CH, edge, 0)

        for r in range(R):
            pltpu.sync_copy(uidx.at[r, w], uidx_v)
            pltpu.sync_copy(qidx.at[r, w], qidx_v)
            start(0, ubuf_a, qbuf_a, usem_a, qsem_a)

            def pair(pp, c):
                ja = 2 * pp
                start(ja + 1, ubuf_b, qbuf_b, usem_b, qsem_b)
                waitbufs(ubuf_a, qbuf_a, usem_a, qsem_a)
                compute(ja, ubuf_a, qbuf_a)

                @pl.when(pp < _NCH // 2 - 1)
                def _():
                    start(ja + 2, ubuf_a, qbuf_a, usem_a, qsem_a)
                waitbufs(ubuf_b, qbuf_b, usem_b, qsem_b)
                compute(ja + 1, ubuf_b, qbuf_b)
                return c
            lax.fori_loop(0, _NCH // 2, pair, 0)
            pltpu.sync_copy(dots_v, out.at[r, pl.ds(w * _EPW, _EPW)])

    return decode


def _sage_body(nsum_ref, deg_ref, feat_ref, Wn_ref, Ws_ref, b_ref, out_ref):
    feat = feat_ref[...]
    for r in range(R):
        deg = jnp.maximum(deg_ref[r, :, 0:1], 1.0)
        nmean = nsum_ref[r] / deg
        acc = jnp.dot(nmean, Wn_ref[r], preferred_element_type=jnp.float32)
        acc = acc + jnp.dot(feat, Ws_ref[r], preferred_element_type=jnp.float32)
        acc = acc + b_ref[0, r * H:(r + 1) * H][None, :]
        out_ref[:, r * H:(r + 1) * H] = jnp.maximum(acc, 0.0)


def _sage_transform(nsum, deg, feat, Wn, Ws, b, n_rows, blk):
    dn = nsum.shape[-1]
    df = feat.shape[-1]
    return pl.pallas_call(
        _sage_body,
        grid=(n_rows // blk,),
        in_specs=[
            pl.BlockSpec((R, blk, dn), lambda i: (0, i, 0)),
            pl.BlockSpec((R, blk, 16), lambda i: (0, i, 0)),
            pl.BlockSpec((blk, df), lambda i: (i, 0)),
            pl.BlockSpec((R, dn, H), lambda i: (0, 0, 0)),
            pl.BlockSpec((R, df, H), lambda i: (0, 0, 0)),
            pl.BlockSpec((1, R * H), lambda i: (0, 0)),
        ],
        out_specs=pl.BlockSpec((blk, R * H), lambda i: (i, 0)),
        out_shape=jax.ShapeDtypeStruct((n_rows, R * H), jnp.float32),
        interpret=INTERPRET,
    )(nsum, deg, feat, Wn, Ws, b)


def _qall_body(rm_ref, bdec_ref, out_ref):
    out_ref[...] = jnp.dot(rm_ref[...], bdec_ref[...],
                           preferred_element_type=jnp.float32)


def _qall(res_movie, bdec):
    blk = 1000
    return pl.pallas_call(
        _qall_body,
        grid=(N_MOVIES // blk,),
        in_specs=[
            pl.BlockSpec((blk, D), lambda i: (i, 0)),
            pl.BlockSpec((D, R * D), lambda i: (0, 0)),
        ],
        out_specs=pl.BlockSpec((blk, R * D), lambda i: (i, 0)),
        out_shape=jax.ShapeDtypeStruct((N_MOVIES, R * D), jnp.float32),
        interpret=INTERPRET,
    )(res_movie, bdec)


def _lsm_body(dots_ref, out_ref):
    x = dots_ref[0]
    lane = lax.broadcasted_iota(jnp.int32, x.shape, 1)
    valid = lane < R
    xm = jnp.where(valid, x, -jnp.inf)
    m = jnp.max(xm, axis=1, keepdims=True)
    ex = jnp.where(valid, jnp.exp(x - m), 0.0)
    lse = jnp.log(jnp.sum(ex, axis=1, keepdims=True))
    out_ref[0] = jnp.where(valid, x - m - lse, 0.0)


def _log_softmax(dots):
    blk = 2048
    return pl.pallas_call(
        _lsm_body,
        grid=(R, E2 // blk),
        in_specs=[pl.BlockSpec((1, blk, 16), lambda r, i: (r, i, 0))],
        out_specs=pl.BlockSpec((1, blk, 16), lambda r, i: (r, i, 0)),
        out_shape=jax.ShapeDtypeStruct((R, E2, 16), jnp.float32),
        interpret=INTERPRET,
    )(dots)


def _pad_edges(idx, fill):
    pad = jnp.full((R, E2 - E), fill, jnp.int32)
    return jnp.concatenate([idx, pad], axis=1)


def kernel(src, dst, ufeats, mfeats, Wn_u, Ws_u, b_u, Wn_m, Ws_m, b_m, decoders):
    src0 = _pad_edges(src, 0)
    dst0 = _pad_edges(dst, 0)
    srcm1 = _pad_edges(src, -1)
    dstm1 = _pad_edges(dst, -1)

    agg_u = _make_agg(DM, N_USERS, 2)
    nu_sum, deg_u = agg_u(
        mfeats,
        dst0.reshape(R, 16, 10, 128), srcm1.reshape(R, 16, 10, 128),
        jnp.zeros((N_USERS // 16, DM), jnp.float32),
        jnp.zeros((N_USERS // 16, 16), jnp.float32))
    agg_m = _make_agg(DU, N_MOVIES, 1)
    nm_sum, deg_m = agg_m(
        ufeats,
        src0.reshape(R, 16, 10, 128), dstm1.reshape(R, 16, 10, 128),
        jnp.zeros((N_MOVIES // 16, DU), jnp.float32),
        jnp.zeros((N_MOVIES // 16, 16), jnp.float32))

    res_user = _sage_transform(nu_sum, deg_u, ufeats, Wn_m, Ws_m,
                               b_m.reshape(1, R * H), N_USERS, 1000)
    res_movie = _sage_transform(nm_sum, deg_m, mfeats, Wn_u, Ws_u,
                                b_u.reshape(1, R * H), N_MOVIES, 1000)
    bdec = decoders.transpose(2, 0, 1).reshape(D, R * D)
    qall = _qall(res_movie, bdec)

    decode = _make_decode()
    dots = decode(res_user, qall, src0.reshape(R, 32, _EPW),
                  dst0.reshape(R, 32, _EPW))

    out = _log_softmax(dots)
    return tuple(out[r, :E, :R] for r in range(R))
